# speedup vs baseline: 1.1257x; 1.1257x over previous
import jax
import jax.numpy as jnp
from jax import lax
from jax.experimental import pallas as pl
from jax.experimental.pallas import tpu as pltpu
from jax.experimental.pallas import tpu_sc as plsc

B, S, D = 4, 8192, 2048
L = 16
NC, NS = 2, 16
NW = NC * NS
COLS = (B * D) // NW
DW = COLS // L
WPB = D // COLS
CH = 128
NPAIR = S // (2 * CH)


def _argmax_body(x_hbm, out_hbm, buf0, buf1, idxbuf, sem0, sem1):
    wid = lax.axis_index("s") * NC + lax.axis_index("c")
    b = wid // WPB
    d0 = (wid % WPB) * COLS

    def src(c):
        return x_hbm.at[b, pl.ds(c * CH, CH), pl.ds(d0, COLS)]

    pltpu.async_copy(src(0), buf0, sem0)
    pltpu.async_copy(src(1), buf1, sem1)

    def scan_chunk(buf, base, carry):
        def s_body(s, carry):
            vals, idxs = carry
            svec = jnp.full((L,), base + s, dtype=jnp.int32)
            nv, ni = [], []
            for g in range(DW):
                v = buf[s, pl.ds(g * L, L)]
                m = v > vals[g]
                nv.append(jnp.where(m, v, vals[g]))
                ni.append(jnp.where(m, svec, idxs[g]))
            return (tuple(nv), tuple(ni))

        return lax.fori_loop(0, CH, s_body, carry)

    neg = jnp.full((L,), -jnp.inf, dtype=jnp.float32)
    zero = jnp.zeros((L,), dtype=jnp.int32)
    carry = (tuple(neg for _ in range(DW)), tuple(zero for _ in range(DW)))

    def pair_body(p, carry):
        c0 = 2 * p
        pltpu.make_async_copy(src(c0), buf0, sem0).wait()
        carry = scan_chunk(buf0, c0 * CH, carry)

        @pl.when(p < NPAIR - 1)
        def _():
            pltpu.async_copy(src(c0 + 2), buf0, sem0)

        pltpu.make_async_copy(src(c0 + 1), buf1, sem1).wait()
        carry = scan_chunk(buf1, (c0 + 1) * CH, carry)

        @pl.when(p < NPAIR - 1)
        def _():
            pltpu.async_copy(src(c0 + 3), buf1, sem1)

        return carry

    carry = lax.fori_loop(0, NPAIR, pair_body, carry)
    _, idxs = carry
    for g in range(DW):
        idxbuf[pl.ds(g * L, L)] = idxs[g]
    pltpu.sync_copy(idxbuf, out_hbm.at[b, pl.ds(d0, COLS)])


def kernel(x):
    mesh = plsc.VectorSubcoreMesh(
        core_axis_name="c", subcore_axis_name="s",
        num_cores=NC, num_subcores=NS,
    )
    f = pl.kernel(
        _argmax_body,
        out_type=jax.ShapeDtypeStruct((B, D), jnp.int32),
        mesh=mesh,
        scratch_types=[
            pltpu.VMEM((CH, COLS), jnp.float32),
            pltpu.VMEM((CH, COLS), jnp.float32),
            pltpu.VMEM((COLS,), jnp.int32),
            pltpu.SemaphoreType.DMA,
            pltpu.SemaphoreType.DMA,
        ],
    )
    return f(x)
